# baseline (speedup 1.0000x reference)
import functools

import jax
import jax.numpy as jnp
from jax import lax
from jax.experimental import pallas as pl
from jax.experimental.pallas import tpu as pltpu, tpu_sc as plsc

_B = 1024
_U = 50
_P = 20
_D = 32
_N = _B * _U

_NC = 2
_NS = 16
_NW = _NC * _NS
_ROWS_W = _N // _NW
_BW = _B // _NW
_C = 400
_CB = _C // _U
_NCHUNK = _ROWS_W // _C
_TB = _B // 128
_UNITS_W = _P * _TB // _NW

_mesh = plsc.VectorSubcoreMesh(core_axis_name="c", subcore_axis_name="s")


@functools.partial(
    pl.kernel,
    out_type=jax.ShapeDtypeStruct((_N * _P,), jnp.int32),
    mesh=_mesh,
    compiler_params=pltpu.CompilerParams(use_tc_tiling_on_sc=True,
                                         needs_layout_passes=False),
    scratch_types=[
        pltpu.VMEM((56, 128), jnp.int32),
        pltpu.VMEM((56, 128), jnp.int32),
        pltpu.VMEM((50 * 128,), jnp.int32),
        pltpu.VMEM((50 * 128,), jnp.int32),
        pltpu.SemaphoreType.DMA,
        pltpu.SemaphoreType.DMA,
        pltpu.SemaphoreType.DMA,
        pltpu.SemaphoreType.DMA,
    ],
)
def _detile(idx_hbm, out_hbm, i0, i1, o0, o1, si0, si1, so0, so1):
    wid = lax.axis_index("s") * _NC + lax.axis_index("c")
    ivs = (i0, i1)
    ovs = (o0, o1)
    sis = (si0, si1)
    sos = (so0, so1)
    lane16 = jnp.arange(16, dtype=jnp.int32)

    def stage(i, b):
        uid = wid * _UNITS_W + i
        g = uid // _TB
        tb = uid % _TB
        return pltpu.async_copy(
            idx_hbm.at[g, :, pl.ds(tb * 128, 128)],
            ivs[b].at[pl.ds(0, _U)], sis[b])

    def repack(i, b):
        iv = ivs[b]
        ov = ovs[b]

        def u_body(u, _):
            def bb_body(bb, _):
                v = iv[u, pl.ds(bb * 16, 16)]
                dst = (bb * 16 + lane16) * _U + u
                plsc.store_scatter(ov, [dst], v)
                return 0

            lax.fori_loop(0, 8, bb_body, 0)
            return 0

        lax.fori_loop(0, _U, u_body, 0)

    def flush(i, b):
        uid = wid * _UNITS_W + i
        g = uid // _TB
        tb = uid % _TB
        return pltpu.async_copy(
            ovs[b], out_hbm.at[pl.ds(g * _N + tb * 128 * _U, 128 * _U)],
            sos[b])

    st = [None, None]
    fl = [None, None]
    st[0] = stage(0, 0)
    for i in range(_UNITS_W):
        b = i & 1
        nb = 1 - b
        if i + 1 < _UNITS_W:
            st[nb] = stage(i + 1, nb)
        st[b].wait()
        if fl[b] is not None:
            fl[b].wait()
        repack(i, b)
        fl[b] = flush(i, b)
    for d in fl:
        if d is not None:
            d.wait()


_V = 100000
_VT = 768
_VTAIL = _V // 128 - _VT


@functools.partial(
    pl.kernel,
    out_type=jax.ShapeDtypeStruct((_V * _D,), jnp.float32),
    mesh=_mesh,
    compiler_params=pltpu.CompilerParams(use_tc_tiling_on_sc=True,
                                         needs_layout_passes=False),
    scratch_types=[
        pltpu.VMEM((_D, 128), jnp.float32),
        pltpu.VMEM((_D, 128), jnp.float32),
        pltpu.VMEM((128 * _D,), jnp.float32),
        pltpu.VMEM((128 * _D,), jnp.float32),
        pltpu.SemaphoreType.DMA,
        pltpu.SemaphoreType.DMA,
        pltpu.SemaphoreType.DMA,
        pltpu.SemaphoreType.DMA,
    ],
)
def _table_prep(wt_hbm, rem_hbm, out_hbm, i0, i1, o0, o1, si0, si1, so0, so1):
    wid = lax.axis_index("s") * _NC + lax.axis_index("c")
    ivs = (i0, i1)
    ovs = (o0, o1)
    sis = (si0, si1)
    sos = (so0, so1)
    lane16 = jnp.arange(16, dtype=jnp.int32)
    zero16 = jnp.zeros((16,), jnp.float32)

    def stage(i, b):
        c0 = (i * _NW + wid) * 128
        return pltpu.async_copy(wt_hbm.at[:, pl.ds(c0, 128)], ivs[b], sis[b])

    def repack(b, ncols):
        iv = ivs[b]
        ov = ovs[b]

        def d_body(d, _):
            def cb_body(cb, _):
                v = iv[d, pl.ds(cb * 16, 16)]
                dst = (cb * 16 + lane16) * _D + d
                plsc.store_scatter(ov, [dst], v)
                return 0

            lax.fori_loop(0, ncols // 16, cb_body, 0)
            return 0

        lax.fori_loop(0, _D, d_body, 0)

    def flush(i, b):
        c0 = (i * _NW + wid) * 128
        return pltpu.async_copy(
            ovs[b], out_hbm.at[pl.ds(c0 * _D, 128 * _D)], sos[b])

    st = [None, None]
    fl = [None, None]
    st[0] = stage(0, 0)
    for i in range(_VT // _NW):
        b = i & 1
        nb = 1 - b
        if i + 1 < _VT // _NW:
            st[nb] = stage(i + 1, nb)
        st[b].wait()
        if fl[b] is not None:
            fl[b].wait()
        repack(b, 128)

        @pl.when(jnp.logical_and(wid == 0, i == 0))
        def _():
            ovs[b][pl.ds(0, 16)] = zero16
            ovs[b][pl.ds(16, 16)] = zero16

        fl[b] = flush(i, b)
    for d in fl:
        if d is not None:
            d.wait()

    @pl.when(wid < _VTAIL)
    def _():
        c0 = (_VT + wid) * 128
        pltpu.sync_copy(wt_hbm.at[:, pl.ds(c0, 128)], ivs[0])
        repack(0, 128)
        pltpu.sync_copy(ovs[0], out_hbm.at[pl.ds(c0 * _D, 128 * _D)])

    @pl.when(wid == _VTAIL)
    def _():
        pltpu.sync_copy(rem_hbm, ovs[0].at[pl.ds(0, 32 * _D)])
        pltpu.sync_copy(ovs[0].at[pl.ds(0, 32 * _D)],
                        out_hbm.at[pl.ds((_V - 32) * _D, 32 * _D)])


@functools.partial(
    pl.kernel,
    out_type=jax.ShapeDtypeStruct((_B, _U, _D), jnp.float32),
    mesh=_mesh,
    compiler_params=pltpu.CompilerParams(use_tc_tiling_on_sc=False,
                                         needs_layout_passes=False),
    scratch_types=[
        pltpu.VMEM((_P, _C), jnp.int32),
        pltpu.VMEM((_P, _C), jnp.int32),
        pltpu.VMEM((_C, _D), jnp.float32),
        pltpu.VMEM((_C, _D), jnp.float32),
        pltpu.SemaphoreType.DMA,
        pltpu.SemaphoreType.DMA,
        pltpu.SemaphoreType.DMA,
        pltpu.SemaphoreType.DMA,
        pltpu.SemaphoreType.DMA,
        pltpu.SemaphoreType.DMA,
    ],
)
def _gather_sum(idx_hbm, table_hbm, out_hbm, it0, it1, a0, a1,
                si0, si1, sg0, sg1, so0, so1):
    wid = lax.axis_index("s") * _NC + lax.axis_index("c")
    w_row = wid * _ROWS_W
    its = (it0, it1)
    accs = (a0, a1)
    sis = (si0, si1)
    sgs = (sg0, sg1)
    sos = (so0, so1)

    zero16 = jnp.zeros((16,), jnp.float32)

    def stage_idx(k, b):
        base = w_row + k * _C
        return [
            pltpu.async_copy(idx_hbm.at[pl.ds(g * _N + base, _C)],
                             its[b].at[g], sis[b])
            for g in range(_P)
        ]

    def zero_acc(b):
        a = accs[b]

        def z_body(j, _):
            a[j, pl.ds(0, 16)] = zero16
            a[j, pl.ds(16, 16)] = zero16
            return 0

        lax.fori_loop(0, _C, z_body, 0)

    def fire(b):
        return [
            pltpu.async_copy(table_hbm.at[its[b].at[g]], accs[b], sgs[b],
                             add=True)
            for g in range(_P)
        ]

    def fire_out(k, b):
        b0 = wid * _BW + k * _CB
        return [
            pltpu.async_copy(accs[b].at[pl.ds(q * _U, _U)],
                             out_hbm.at[b0 + q], sos[b])
            for q in range(_CB)
        ]

    descs = [None, None]
    out_descs = [None, None]
    st = [None, None]

    st[0] = stage_idx(0, 0)
    zero_acc(0)
    for d in st[0]:
        d.wait()
    descs[0] = fire(0)
    for k in range(_NCHUNK):
        b = k & 1
        nb = 1 - b
        if k + 1 < _NCHUNK:
            st[nb] = stage_idx(k + 1, nb)
            if k + 1 >= 2:
                for d in out_descs[nb]:
                    d.wait()
            zero_acc(nb)
            for d in st[nb]:
                d.wait()
            descs[nb] = fire(nb)
        for d in descs[b]:
            d.wait()
        out_descs[b] = fire_out(k, b)
    for b in ((_NCHUNK - 1) & 1, (_NCHUNK - 2) & 1):
        for d in out_descs[b]:
            d.wait()


def kernel(prev_cmd, num_cmd, ctype_emb_weight):
    del num_cmd
    idx = _detile(prev_cmd.astype(jnp.int32).transpose(2, 1, 0))
    rem = ctype_emb_weight[_V - 32:].reshape(-1)
    table = _table_prep(ctype_emb_weight.transpose(1, 0), rem).reshape(_V, _D)
    return _gather_sum(idx, table)


# speedup vs baseline: 1.2006x; 1.2006x over previous
import functools

import jax
import jax.numpy as jnp
from jax import lax
from jax.experimental import pallas as pl
from jax.experimental.pallas import tpu as pltpu, tpu_sc as plsc

_B = 1024
_U = 50
_P = 20
_D = 32
_N = _B * _U

_NC = 2
_NS = 16
_NW = _NC * _NS
_ROWS_W = _N // _NW
_BW = _B // _NW
_C = 400
_CB = _C // _U
_NCHUNK = _ROWS_W // _C
_TB = _B // 128
_UNITS_W = _P * _TB // _NW

_mesh = plsc.VectorSubcoreMesh(core_axis_name="c", subcore_axis_name="s")


@functools.partial(
    pl.kernel,
    out_type=jax.ShapeDtypeStruct((_N * _P,), jnp.int32),
    mesh=_mesh,
    compiler_params=pltpu.CompilerParams(use_tc_tiling_on_sc=True,
                                         needs_layout_passes=False),
    scratch_types=[
        pltpu.VMEM((56, 128), jnp.int32),
        pltpu.VMEM((56, 128), jnp.int32),
        pltpu.VMEM((50 * 128,), jnp.int32),
        pltpu.VMEM((50 * 128,), jnp.int32),
        pltpu.SemaphoreType.DMA,
        pltpu.SemaphoreType.DMA,
        pltpu.SemaphoreType.DMA,
        pltpu.SemaphoreType.DMA,
    ],
)
def _detile(idx_hbm, out_hbm, i0, i1, o0, o1, si0, si1, so0, so1):
    wid = lax.axis_index("s") * _NC + lax.axis_index("c")
    ivs = (i0, i1)
    ovs = (o0, o1)
    sis = (si0, si1)
    sos = (so0, so1)
    lane16 = jnp.arange(16, dtype=jnp.int32)

    def stage(i, b):
        uid = wid * _UNITS_W + i
        g = uid // _TB
        tb = uid % _TB
        return pltpu.async_copy(
            idx_hbm.at[g, :, pl.ds(tb * 128, 128)],
            ivs[b].at[pl.ds(0, _U)], sis[b])

    def repack(i, b):
        iv = ivs[b]
        ov = ovs[b]

        def u_body(u, _):
            def bb_body(bb, _):
                v = iv[u, pl.ds(bb * 16, 16)]
                dst = (bb * 16 + lane16) * _U + u
                plsc.store_scatter(ov, [dst], v)
                return 0

            lax.fori_loop(0, 8, bb_body, 0)
            return 0

        lax.fori_loop(0, _U, u_body, 0)

    def flush(i, b):
        uid = wid * _UNITS_W + i
        g = uid // _TB
        tb = uid % _TB
        return pltpu.async_copy(
            ovs[b], out_hbm.at[pl.ds(g * _N + tb * 128 * _U, 128 * _U)],
            sos[b])

    st = [None, None]
    fl = [None, None]
    st[0] = stage(0, 0)
    for i in range(_UNITS_W):
        b = i & 1
        nb = 1 - b
        if i + 1 < _UNITS_W:
            st[nb] = stage(i + 1, nb)
        st[b].wait()
        if fl[b] is not None:
            fl[b].wait()
        repack(i, b)
        fl[b] = flush(i, b)
    for d in fl:
        if d is not None:
            d.wait()


@functools.partial(
    pl.kernel,
    out_type=jax.ShapeDtypeStruct((_B, _U, _D), jnp.float32),
    mesh=_mesh,
    compiler_params=pltpu.CompilerParams(use_tc_tiling_on_sc=False,
                                         needs_layout_passes=False),
    scratch_types=[
        pltpu.VMEM((_P, _C), jnp.int32),
        pltpu.VMEM((_P, _C), jnp.int32),
        pltpu.VMEM((_C, _D), jnp.float32),
        pltpu.VMEM((_C, _D), jnp.float32),
        pltpu.SemaphoreType.DMA,
        pltpu.SemaphoreType.DMA,
        pltpu.SemaphoreType.DMA,
        pltpu.SemaphoreType.DMA,
        pltpu.SemaphoreType.DMA,
        pltpu.SemaphoreType.DMA,
    ],
)
def _gather_sum(idx_hbm, table_hbm, out_hbm, it0, it1, a0, a1,
                si0, si1, sg0, sg1, so0, so1):
    wid = lax.axis_index("s") * _NC + lax.axis_index("c")
    w_row = wid * _ROWS_W
    its = (it0, it1)
    accs = (a0, a1)
    sis = (si0, si1)
    sgs = (sg0, sg1)
    sos = (so0, so1)

    zero16 = jnp.zeros((16,), jnp.float32)

    def stage_idx(k, b):
        base = w_row + k * _C
        return [
            pltpu.async_copy(idx_hbm.at[pl.ds(g * _N + base, _C)],
                             its[b].at[g], sis[b])
            for g in range(_P)
        ]

    def zero_acc(b):
        a = accs[b]

        def z_body(j, _):
            a[j, pl.ds(0, 16)] = zero16
            a[j, pl.ds(16, 16)] = zero16
            return 0

        lax.fori_loop(0, _C, z_body, 0)

    def fire(b):
        return [
            pltpu.async_copy(table_hbm.at[its[b].at[g]], accs[b], sgs[b],
                             add=True)
            for g in range(_P)
        ]

    def fire_out(k, b):
        b0 = wid * _BW + k * _CB
        return [
            pltpu.async_copy(accs[b].at[pl.ds(q * _U, _U)],
                             out_hbm.at[b0 + q], sos[b])
            for q in range(_CB)
        ]

    descs = [None, None]
    out_descs = [None, None]
    st = [None, None]

    st[0] = stage_idx(0, 0)
    zero_acc(0)
    for d in st[0]:
        d.wait()
    descs[0] = fire(0)
    for k in range(_NCHUNK):
        b = k & 1
        nb = 1 - b
        if k + 1 < _NCHUNK:
            st[nb] = stage_idx(k + 1, nb)
            if k + 1 >= 2:
                for d in out_descs[nb]:
                    d.wait()
            zero_acc(nb)
            for d in st[nb]:
                d.wait()
            descs[nb] = fire(nb)
        for d in descs[b]:
            d.wait()
        out_descs[b] = fire_out(k, b)
    for b in ((_NCHUNK - 1) & 1, (_NCHUNK - 2) & 1):
        for d in out_descs[b]:
            d.wait()


def kernel(prev_cmd, num_cmd, ctype_emb_weight):
    del num_cmd
    table = ctype_emb_weight.at[0].set(0.0)
    idx = _detile(prev_cmd.astype(jnp.int32).transpose(2, 1, 0))
    return _gather_sum(idx, table)
